# baseline (speedup 1.0000x reference)
import functools

import jax
import jax.numpy as jnp
from jax import lax
from jax.experimental import pallas as pl
from jax.experimental.pallas import tpu as pltpu
from jax.experimental.pallas import tpu_sc as plsc

NC = 2
NS = 16
NW = NC * NS
LANES = 16

_VB = 4096
_VBQ = _VB // 4
_HI = 0xFFFF0000


def _tc_pack_table(table):
    V, D = table.shape
    grid = -(-V // _VB)

    def body(in_ref, out_ref):
        x = in_ref[...]
        u = lax.bitcast_convert_type(x, jnp.uint32)
        rb = (u + jnp.uint32(0x7FFF) + ((u >> 16) & jnp.uint32(1))) >> 16
        w = rb[:D // 2, :] | (rb[D // 2:, :] << 16)
        wt = w.T
        out_ref[...] = jnp.concatenate(
            [wt[k * _VBQ:(k + 1) * _VBQ] for k in range(4)], axis=1)

    out = pl.pallas_call(
        body,
        grid=(grid,),
        in_specs=[pl.BlockSpec((D, _VB), lambda i: (0, i))],
        out_specs=pl.BlockSpec((_VBQ, 2 * D), lambda i: (i, 0)),
        out_shape=jax.ShapeDtypeStruct((grid * _VBQ, 2 * D), jnp.uint32),
    )(table.T)
    return out.reshape(grid * _VB, D // 2)


def _map_ids(v):
    return (v & -_VB) + ((v & (_VBQ - 1)) << 2) + ((v >> (_VBQ.bit_length() - 1)) & 3)


def _sc_logits(B, P, N, D, W, GCH, UNROLL):
    BPT = B // NW
    NCHUNK = BPT // W
    NBUF = 2
    WP, WN = W * P, W * N
    DW = D // 2

    mesh = plsc.VectorSubcoreMesh(core_axis_name="c", subcore_axis_name="s")

    scratch = []
    for _ in range(NBUF):
        scratch += [
            pltpu.VMEM((W,), jnp.int32),
            pltpu.VMEM((WP,), jnp.int32),
            pltpu.VMEM((WN,), jnp.int32),
            pltpu.VMEM((W, DW), jnp.uint32),
            pltpu.VMEM((WP, DW), jnp.uint32),
            pltpu.VMEM((WN, DW), jnp.uint32),
            pltpu.SemaphoreType.DMA,
            pltpu.SemaphoreType.DMA,
        ]
    scratch += [
        pltpu.VMEM((BPT * P,), jnp.float32),
        pltpu.VMEM((BPT * N,), jnp.float32),
    ]

    @functools.partial(
        pl.kernel,
        out_type=(
            jax.ShapeDtypeStruct((B * P,), jnp.float32),
            jax.ShapeDtypeStruct((B * N,), jnp.float32),
        ),
        mesh=mesh,
        scratch_types=scratch,
        compiler_params=pltpu.CompilerParams(needs_layout_passes=False,
                                             use_tc_tiling_on_sc=False),
    )
    def k(inp_ids, near_ids, neg_ids, table, out_near, out_neg, *s):
        bufs = [s[i * 8:(i + 1) * 8] for i in range(NBUF)]
        ln_all, lg_all = s[NBUF * 8], s[NBUF * 8 + 1]
        wid = lax.axis_index("s") * NC + lax.axis_index("c")
        base = wid * BPT
        LAST = NCHUNK - 1

        def fire_idx(c, r):
            idx_i, idx_p, idx_n = bufs[r][0:3]
            isem = bufs[r][7]
            b0 = base + c * W
            pltpu.async_copy(inp_ids.at[pl.ds(b0, W)], idx_i, isem)
            pltpu.async_copy(near_ids.at[pl.ds(b0 * P, WP)], idx_p, isem)
            pltpu.async_copy(neg_ids.at[pl.ds(b0 * N, WN)], idx_n, isem)

        def wait_idx(r):
            idx_i, idx_p, idx_n = bufs[r][0:3]
            isem = bufs[r][7]
            pltpu.make_async_copy(inp_ids.at[pl.ds(0, W)], idx_i, isem).wait()
            pltpu.make_async_copy(near_ids.at[pl.ds(0, WP)], idx_p, isem).wait()
            pltpu.make_async_copy(neg_ids.at[pl.ds(0, WN)], idx_n, isem).wait()

        def fire_gathers(r):
            idx_i, idx_p, idx_n, rows_i, rows_p, rows_n, gsem, _ = bufs[r]
            pltpu.async_copy(table.at[idx_i], rows_i, gsem)
            for o in range(0, WP, GCH):
                pltpu.async_copy(table.at[idx_p.at[pl.ds(o, GCH)]],
                                 rows_p.at[pl.ds(o, GCH)], gsem)
            for o in range(0, WN, GCH):
                pltpu.async_copy(table.at[idx_n.at[pl.ds(o, GCH)]],
                                 rows_n.at[pl.ds(o, GCH)], gsem)

        def drain_gathers(r):
            idx_i, idx_p, idx_n, rows_i, rows_p, rows_n, gsem, _ = bufs[r]
            pltpu.make_async_copy(table.at[idx_i], rows_i, gsem).wait()
            for o in range(0, WP, GCH):
                pltpu.make_async_copy(table.at[idx_p.at[pl.ds(o, GCH)]],
                                      rows_p.at[pl.ds(o, GCH)], gsem).wait()
            for o in range(0, WN, GCH):
                pltpu.make_async_copy(table.at[idx_n.at[pl.ds(o, GCH)]],
                                      rows_n.at[pl.ds(o, GCH)], gsem).wait()

        lane = lax.iota(jnp.int32, LANES)
        last_lane = lane == (LANES - 1)

        def unpack(w):
            return (plsc.bitcast(w << 16, jnp.float32),
                    plsc.bitcast(w & jnp.uint32(_HI), jnp.float32))

        def compute(c, r):
            rows_i, rows_p, rows_n = bufs[r][3:6]
            for b in range(W):
                ivecs = []
                for j in range(DW // LANES):
                    ivecs += unpack(rows_i[b, pl.ds(j * LANES, LANES)])

                def dots(K, rows, out, obase):
                    @plsc.parallel_loop(0, K, 1, unroll=UNROLL)
                    def _(kk):
                        row = b * K + kk
                        acc = None
                        for j in range(DW // LANES):
                            a, hi = unpack(rows[row, pl.ds(j * LANES, LANES)])
                            t = a * ivecs[2 * j] + hi * ivecs[2 * j + 1]
                            acc = t if acc is None else acc + t
                        s_ = plsc.cumsum(acc)
                        oidx = jnp.full((LANES,), obase + kk, jnp.int32)
                        plsc.store_scatter(out, [oidx], s_, mask=last_lane)

                dots(P, rows_p, ln_all, (c * W + b) * P)
                dots(N, rows_n, lg_all, (c * W + b) * N)

        fire_idx(0, 0)
        wait_idx(0)
        fire_gathers(0)
        fire_idx(1, 1)

        def step(i, _):
            for r in range(NBUF):
                c = i * NBUF + r
                drain_gathers(r)
                fire_idx(jnp.minimum(c + 2, LAST), r)
                wait_idx(r ^ 1)
                fire_gathers(r ^ 1)
                compute(c, r)
            return 0

        lax.fori_loop(0, NCHUNK // NBUF, step, 0)
        drain_gathers(0)
        wait_idx(1)

        pltpu.sync_copy(ln_all, out_near.at[pl.ds(base * P, BPT * P)])
        pltpu.sync_copy(lg_all, out_neg.at[pl.ds(base * N, BPT * N)])

    return k


def _tc_loss(ln, lg):
    B, P = ln.shape
    N = lg.shape[1]
    BLK = 2048

    def body(ln_ref, lg_ref, out_ref):
        def lsig(x):
            return jnp.minimum(x, 0.0) - jnp.log1p(jnp.exp(-jnp.abs(x)))
        out_ref[...] = -(lsig(ln_ref[...]).sum(axis=1)
                         + lsig(-lg_ref[...]).sum(axis=1))

    return pl.pallas_call(
        body,
        grid=(B // BLK,),
        in_specs=[
            pl.BlockSpec((BLK, P), lambda i: (i, 0)),
            pl.BlockSpec((BLK, N), lambda i: (i, 0)),
        ],
        out_specs=pl.BlockSpec((BLK,), lambda i: (i,)),
        out_shape=jax.ShapeDtypeStruct((B,), jnp.float32),
    )(ln, lg)


def kernel(input_wordids, near_wordids, neg_wordids, input_weight):
    B, P = near_wordids.shape
    N = neg_wordids.shape[1]
    V, D = input_weight.shape
    W = 8
    GCH = 80
    UNROLL = 5

    ids = _map_ids(input_wordids.astype(jnp.int32))
    near = _map_ids(near_wordids.reshape(B * P).astype(jnp.int32))
    neg = _map_ids(neg_wordids.reshape(B * N).astype(jnp.int32))
    table_pk = _tc_pack_table(input_weight)

    ln, lg = _sc_logits(B, P, N, D, W, GCH, UNROLL)(ids, near, neg, table_pk)
    return _tc_loss(ln.reshape(B, P), lg.reshape(B, N))


# speedup vs baseline: 1.1502x; 1.1502x over previous
import functools

import jax
import jax.numpy as jnp
from jax import lax
from jax.experimental import pallas as pl
from jax.experimental.pallas import tpu as pltpu
from jax.experimental.pallas import tpu_sc as plsc

NC = 2
NS = 16
NW = NC * NS
LANES = 16

_VB = 16384
_VBQ = _VB // 4
_HI = 0xFFFF0000


def _tc_pack_table(table):
    V, D = table.shape
    grid = -(-V // _VB)

    def body(in_ref, out_ref):
        x = in_ref[...]
        u = lax.bitcast_convert_type(x, jnp.uint32)
        rb = (u + jnp.uint32(0x7FFF) + ((u >> 16) & jnp.uint32(1))) >> 16
        w = rb[:D // 2, :] | (rb[D // 2:, :] << 16)
        wt = w.T
        out_ref[...] = jnp.concatenate(
            [wt[k * _VBQ:(k + 1) * _VBQ] for k in range(4)], axis=1)

    out = pl.pallas_call(
        body,
        grid=(grid,),
        in_specs=[pl.BlockSpec((D, _VB), lambda i: (0, i))],
        out_specs=pl.BlockSpec((_VBQ, 2 * D), lambda i: (i, 0)),
        out_shape=jax.ShapeDtypeStruct((grid * _VBQ, 2 * D), jnp.uint32),
    )(table.T)
    return out.reshape(grid * _VB, D // 2)


def _map_ids(v):
    return (v & -_VB) + ((v & (_VBQ - 1)) << 2) + ((v >> (_VBQ.bit_length() - 1)) & 3)


def _sc_logits(B, P, N, D, W, GCH, UNROLL):
    BPT = B // NW
    NCHUNK = BPT // W
    NBUF = 2
    WP, WN = W * P, W * N
    DW = D // 2

    mesh = plsc.VectorSubcoreMesh(core_axis_name="c", subcore_axis_name="s")

    scratch = []
    for _ in range(NBUF):
        scratch += [
            pltpu.VMEM((W,), jnp.int32),
            pltpu.VMEM((WP,), jnp.int32),
            pltpu.VMEM((WN,), jnp.int32),
            pltpu.VMEM((W, DW), jnp.uint32),
            pltpu.VMEM((WP, DW), jnp.uint32),
            pltpu.VMEM((WN, DW), jnp.uint32),
            pltpu.SemaphoreType.DMA,
            pltpu.SemaphoreType.DMA,
        ]
    scratch += [
        pltpu.VMEM((BPT * P,), jnp.float32),
        pltpu.VMEM((BPT * N,), jnp.float32),
    ]

    @functools.partial(
        pl.kernel,
        out_type=(
            jax.ShapeDtypeStruct((B * P,), jnp.float32),
            jax.ShapeDtypeStruct((B * N,), jnp.float32),
        ),
        mesh=mesh,
        scratch_types=scratch,
        compiler_params=pltpu.CompilerParams(needs_layout_passes=False,
                                             use_tc_tiling_on_sc=False),
    )
    def k(inp_ids, near_ids, neg_ids, table, out_near, out_neg, *s):
        bufs = [s[i * 8:(i + 1) * 8] for i in range(NBUF)]
        ln_all, lg_all = s[NBUF * 8], s[NBUF * 8 + 1]
        wid = lax.axis_index("s") * NC + lax.axis_index("c")
        base = wid * BPT
        LAST = NCHUNK - 1

        def fire_idx(c, r):
            idx_i, idx_p, idx_n = bufs[r][0:3]
            isem = bufs[r][7]
            b0 = base + c * W
            pltpu.async_copy(inp_ids.at[pl.ds(b0, W)], idx_i, isem)
            pltpu.async_copy(near_ids.at[pl.ds(b0 * P, WP)], idx_p, isem)
            pltpu.async_copy(neg_ids.at[pl.ds(b0 * N, WN)], idx_n, isem)

        def wait_idx(r):
            idx_i, idx_p, idx_n = bufs[r][0:3]
            isem = bufs[r][7]
            pltpu.make_async_copy(inp_ids.at[pl.ds(0, W)], idx_i, isem).wait()
            pltpu.make_async_copy(near_ids.at[pl.ds(0, WP)], idx_p, isem).wait()
            pltpu.make_async_copy(neg_ids.at[pl.ds(0, WN)], idx_n, isem).wait()

        def fire_gathers(r):
            idx_i, idx_p, idx_n, rows_i, rows_p, rows_n, gsem, _ = bufs[r]
            pltpu.async_copy(table.at[idx_i], rows_i, gsem)
            for o in range(0, WP, GCH):
                pltpu.async_copy(table.at[idx_p.at[pl.ds(o, GCH)]],
                                 rows_p.at[pl.ds(o, GCH)], gsem)
            for o in range(0, WN, GCH):
                pltpu.async_copy(table.at[idx_n.at[pl.ds(o, GCH)]],
                                 rows_n.at[pl.ds(o, GCH)], gsem)

        def drain_gathers(r):
            idx_i, idx_p, idx_n, rows_i, rows_p, rows_n, gsem, _ = bufs[r]
            pltpu.make_async_copy(table.at[idx_i], rows_i, gsem).wait()
            for o in range(0, WP, GCH):
                pltpu.make_async_copy(table.at[idx_p.at[pl.ds(o, GCH)]],
                                      rows_p.at[pl.ds(o, GCH)], gsem).wait()
            for o in range(0, WN, GCH):
                pltpu.make_async_copy(table.at[idx_n.at[pl.ds(o, GCH)]],
                                      rows_n.at[pl.ds(o, GCH)], gsem).wait()

        lane = lax.iota(jnp.int32, LANES)
        last_lane = lane == (LANES - 1)

        def unpack(w):
            return (plsc.bitcast(w << 16, jnp.float32),
                    plsc.bitcast(w & jnp.uint32(_HI), jnp.float32))

        def compute(c, r):
            rows_i, rows_p, rows_n = bufs[r][3:6]
            for b in range(W):
                ivecs = []
                for j in range(DW // LANES):
                    ivecs += unpack(rows_i[b, pl.ds(j * LANES, LANES)])

                def dots(K, rows, out, obase):
                    @plsc.parallel_loop(0, K, 1, unroll=UNROLL)
                    def _(kk):
                        row = b * K + kk
                        acc = None
                        for j in range(DW // LANES):
                            a, hi = unpack(rows[row, pl.ds(j * LANES, LANES)])
                            t = a * ivecs[2 * j] + hi * ivecs[2 * j + 1]
                            acc = t if acc is None else acc + t
                        s_ = plsc.cumsum(acc)
                        oidx = jnp.full((LANES,), obase + kk, jnp.int32)
                        plsc.store_scatter(out, [oidx], s_, mask=last_lane)

                dots(P, rows_p, ln_all, (c * W + b) * P)
                dots(N, rows_n, lg_all, (c * W + b) * N)

        fire_idx(0, 0)
        wait_idx(0)
        fire_gathers(0)
        fire_idx(1, 1)

        def step(i, _):
            for r in range(NBUF):
                c = i * NBUF + r
                drain_gathers(r)
                fire_idx(jnp.minimum(c + 2, LAST), r)
                wait_idx(r ^ 1)
                fire_gathers(r ^ 1)
                compute(c, r)
            return 0

        lax.fori_loop(0, NCHUNK // NBUF, step, 0)
        drain_gathers(0)
        wait_idx(1)

        pltpu.sync_copy(ln_all, out_near.at[pl.ds(base * P, BPT * P)])
        pltpu.sync_copy(lg_all, out_neg.at[pl.ds(base * N, BPT * N)])

    return k


def _tc_loss(ln, lg):
    B, P = ln.shape
    N = lg.shape[1]
    BLK = 2048

    def body(ln_ref, lg_ref, out_ref):
        def lsig(x):
            return jnp.minimum(x, 0.0) - jnp.log1p(jnp.exp(-jnp.abs(x)))
        out_ref[...] = -(lsig(ln_ref[...]).sum(axis=1)
                         + lsig(-lg_ref[...]).sum(axis=1))

    return pl.pallas_call(
        body,
        grid=(B // BLK,),
        in_specs=[
            pl.BlockSpec((BLK, P), lambda i: (i, 0)),
            pl.BlockSpec((BLK, N), lambda i: (i, 0)),
        ],
        out_specs=pl.BlockSpec((BLK,), lambda i: (i,)),
        out_shape=jax.ShapeDtypeStruct((B,), jnp.float32),
    )(ln, lg)


def kernel(input_wordids, near_wordids, neg_wordids, input_weight):
    B, P = near_wordids.shape
    N = neg_wordids.shape[1]
    V, D = input_weight.shape
    W = 8
    GCH = 80
    UNROLL = 5

    ids = _map_ids(input_wordids.astype(jnp.int32))
    near = _map_ids(near_wordids.reshape(B * P).astype(jnp.int32))
    neg = _map_ids(neg_wordids.reshape(B * N).astype(jnp.int32))
    table_pk = _tc_pack_table(input_weight)

    ln, lg = _sc_logits(B, P, N, D, W, GCH, UNROLL)(ids, near, neg, table_pk)
    return _tc_loss(ln.reshape(B, P), lg.reshape(B, N))


# speedup vs baseline: 1.1527x; 1.0022x over previous
import functools

import jax
import jax.numpy as jnp
from jax import lax
from jax.experimental import pallas as pl
from jax.experimental.pallas import tpu as pltpu
from jax.experimental.pallas import tpu_sc as plsc

NC = 2
NS = 16
NW = NC * NS
LANES = 16

_VB = 32768
_VBQ = _VB // 4
_HI = 0xFFFF0000


def _tc_pack_table(table):
    V, D = table.shape
    grid = -(-V // _VB)

    def body(in_ref, out_ref):
        x = in_ref[...]
        u = lax.bitcast_convert_type(x, jnp.uint32)
        rb = (u + jnp.uint32(0x7FFF) + ((u >> 16) & jnp.uint32(1))) >> 16
        w = rb[:D // 2, :] | (rb[D // 2:, :] << 16)
        wt = w.T
        out_ref[...] = jnp.concatenate(
            [wt[k * _VBQ:(k + 1) * _VBQ] for k in range(4)], axis=1)

    out = pl.pallas_call(
        body,
        grid=(grid,),
        in_specs=[pl.BlockSpec((D, _VB), lambda i: (0, i))],
        out_specs=pl.BlockSpec((_VBQ, 2 * D), lambda i: (i, 0)),
        out_shape=jax.ShapeDtypeStruct((grid * _VBQ, 2 * D), jnp.uint32),
    )(table.T)
    return out.reshape(grid * _VB, D // 2)


def _map_ids(v):
    return (v & -_VB) + ((v & (_VBQ - 1)) << 2) + ((v >> (_VBQ.bit_length() - 1)) & 3)


def _sc_logits(B, P, N, D, W, GCH, UNROLL):
    BPT = B // NW
    NCHUNK = BPT // W
    NBUF = 2
    WP, WN = W * P, W * N
    DW = D // 2

    mesh = plsc.VectorSubcoreMesh(core_axis_name="c", subcore_axis_name="s")

    scratch = []
    for _ in range(NBUF):
        scratch += [
            pltpu.VMEM((W,), jnp.int32),
            pltpu.VMEM((WP,), jnp.int32),
            pltpu.VMEM((WN,), jnp.int32),
            pltpu.VMEM((W, DW), jnp.uint32),
            pltpu.VMEM((WP, DW), jnp.uint32),
            pltpu.VMEM((WN, DW), jnp.uint32),
            pltpu.SemaphoreType.DMA,
            pltpu.SemaphoreType.DMA,
        ]
    scratch += [
        pltpu.VMEM((BPT * P,), jnp.float32),
        pltpu.VMEM((BPT * N,), jnp.float32),
    ]

    @functools.partial(
        pl.kernel,
        out_type=(
            jax.ShapeDtypeStruct((B * P,), jnp.float32),
            jax.ShapeDtypeStruct((B * N,), jnp.float32),
        ),
        mesh=mesh,
        scratch_types=scratch,
        compiler_params=pltpu.CompilerParams(needs_layout_passes=False,
                                             use_tc_tiling_on_sc=False),
    )
    def k(inp_ids, near_ids, neg_ids, table, out_near, out_neg, *s):
        bufs = [s[i * 8:(i + 1) * 8] for i in range(NBUF)]
        ln_all, lg_all = s[NBUF * 8], s[NBUF * 8 + 1]
        wid = lax.axis_index("s") * NC + lax.axis_index("c")
        base = wid * BPT
        LAST = NCHUNK - 1

        def fire_idx(c, r):
            idx_i, idx_p, idx_n = bufs[r][0:3]
            isem = bufs[r][7]
            b0 = base + c * W
            pltpu.async_copy(inp_ids.at[pl.ds(b0, W)], idx_i, isem)
            pltpu.async_copy(near_ids.at[pl.ds(b0 * P, WP)], idx_p, isem)
            pltpu.async_copy(neg_ids.at[pl.ds(b0 * N, WN)], idx_n, isem)

        def wait_idx(r):
            idx_i, idx_p, idx_n = bufs[r][0:3]
            isem = bufs[r][7]
            pltpu.make_async_copy(inp_ids.at[pl.ds(0, W)], idx_i, isem).wait()
            pltpu.make_async_copy(near_ids.at[pl.ds(0, WP)], idx_p, isem).wait()
            pltpu.make_async_copy(neg_ids.at[pl.ds(0, WN)], idx_n, isem).wait()

        def fire_gathers(r):
            idx_i, idx_p, idx_n, rows_i, rows_p, rows_n, gsem, _ = bufs[r]
            pltpu.async_copy(table.at[idx_i], rows_i, gsem)
            for o in range(0, WP, GCH):
                pltpu.async_copy(table.at[idx_p.at[pl.ds(o, GCH)]],
                                 rows_p.at[pl.ds(o, GCH)], gsem)
            for o in range(0, WN, GCH):
                pltpu.async_copy(table.at[idx_n.at[pl.ds(o, GCH)]],
                                 rows_n.at[pl.ds(o, GCH)], gsem)

        def drain_gathers(r):
            idx_i, idx_p, idx_n, rows_i, rows_p, rows_n, gsem, _ = bufs[r]
            pltpu.make_async_copy(table.at[idx_i], rows_i, gsem).wait()
            for o in range(0, WP, GCH):
                pltpu.make_async_copy(table.at[idx_p.at[pl.ds(o, GCH)]],
                                      rows_p.at[pl.ds(o, GCH)], gsem).wait()
            for o in range(0, WN, GCH):
                pltpu.make_async_copy(table.at[idx_n.at[pl.ds(o, GCH)]],
                                      rows_n.at[pl.ds(o, GCH)], gsem).wait()

        lane = lax.iota(jnp.int32, LANES)
        last_lane = lane == (LANES - 1)

        def unpack(w):
            return (plsc.bitcast(w << 16, jnp.float32),
                    plsc.bitcast(w & jnp.uint32(_HI), jnp.float32))

        def compute(c, r):
            rows_i, rows_p, rows_n = bufs[r][3:6]
            for b in range(W):
                ivecs = []
                for j in range(DW // LANES):
                    ivecs += unpack(rows_i[b, pl.ds(j * LANES, LANES)])

                def dots(K, rows, out, obase):
                    @plsc.parallel_loop(0, K, 1, unroll=UNROLL)
                    def _(kk):
                        row = b * K + kk
                        acc = None
                        for j in range(DW // LANES):
                            a, hi = unpack(rows[row, pl.ds(j * LANES, LANES)])
                            t = a * ivecs[2 * j] + hi * ivecs[2 * j + 1]
                            acc = t if acc is None else acc + t
                        s_ = plsc.cumsum(acc)
                        oidx = jnp.full((LANES,), obase + kk, jnp.int32)
                        plsc.store_scatter(out, [oidx], s_, mask=last_lane)

                dots(P, rows_p, ln_all, (c * W + b) * P)
                dots(N, rows_n, lg_all, (c * W + b) * N)

        fire_idx(0, 0)
        wait_idx(0)
        fire_gathers(0)
        fire_idx(1, 1)

        def step(i, _):
            for r in range(NBUF):
                c = i * NBUF + r
                drain_gathers(r)
                fire_idx(jnp.minimum(c + 2, LAST), r)
                wait_idx(r ^ 1)
                fire_gathers(r ^ 1)
                compute(c, r)
            return 0

        lax.fori_loop(0, NCHUNK // NBUF, step, 0)
        drain_gathers(0)
        wait_idx(1)

        pltpu.sync_copy(ln_all, out_near.at[pl.ds(base * P, BPT * P)])
        pltpu.sync_copy(lg_all, out_neg.at[pl.ds(base * N, BPT * N)])

    return k


def _tc_loss(ln, lg):
    B, P = ln.shape
    N = lg.shape[1]
    BLK = 2048

    def body(ln_ref, lg_ref, out_ref):
        def lsig(x):
            return jnp.minimum(x, 0.0) - jnp.log1p(jnp.exp(-jnp.abs(x)))
        out_ref[...] = -(lsig(ln_ref[...]).sum(axis=1)
                         + lsig(-lg_ref[...]).sum(axis=1))

    return pl.pallas_call(
        body,
        grid=(B // BLK,),
        in_specs=[
            pl.BlockSpec((BLK, P), lambda i: (i, 0)),
            pl.BlockSpec((BLK, N), lambda i: (i, 0)),
        ],
        out_specs=pl.BlockSpec((BLK,), lambda i: (i,)),
        out_shape=jax.ShapeDtypeStruct((B,), jnp.float32),
    )(ln, lg)


def kernel(input_wordids, near_wordids, neg_wordids, input_weight):
    B, P = near_wordids.shape
    N = neg_wordids.shape[1]
    V, D = input_weight.shape
    W = 8
    GCH = 80
    UNROLL = 5

    ids = _map_ids(input_wordids.astype(jnp.int32))
    near = _map_ids(near_wordids.reshape(B * P).astype(jnp.int32))
    neg = _map_ids(neg_wordids.reshape(B * N).astype(jnp.int32))
    table_pk = _tc_pack_table(input_weight)

    ln, lg = _sc_logits(B, P, N, D, W, GCH, UNROLL)(ids, near, neg, table_pk)
    return _tc_loss(ln.reshape(B, P), lg.reshape(B, N))


# speedup vs baseline: 1.1688x; 1.0140x over previous
import functools

import jax
import jax.numpy as jnp
from jax import lax
from jax.experimental import pallas as pl
from jax.experimental.pallas import tpu as pltpu
from jax.experimental.pallas import tpu_sc as plsc

NC = 2
NS = 16
NW = NC * NS
LANES = 16

_VB = 32768
_VBQ = _VB // 4
_HI = 0xFFFF0000


def _tc_pack_table(table):
    V, D = table.shape
    grid = -(-V // _VB)

    def body(in_ref, out_ref):
        x = in_ref[...]
        u = lax.bitcast_convert_type(x, jnp.uint32)
        rb = (u + jnp.uint32(0x7FFF) + ((u >> 16) & jnp.uint32(1))) >> 16
        w = rb[:D // 2, :] | (rb[D // 2:, :] << 16)
        wt = w.T
        out_ref[...] = jnp.concatenate(
            [wt[k * _VBQ:(k + 1) * _VBQ] for k in range(4)], axis=1)

    out = pl.pallas_call(
        body,
        grid=(grid,),
        in_specs=[pl.BlockSpec((D, _VB), lambda i: (0, i))],
        out_specs=pl.BlockSpec((_VBQ, 2 * D), lambda i: (i, 0)),
        out_shape=jax.ShapeDtypeStruct((grid * _VBQ, 2 * D), jnp.uint32),
    )(table.T)
    return out.reshape(grid * _VB, D // 2)


def _map_ids(v):
    return (v & -_VB) + ((v & (_VBQ - 1)) << 2) + ((v >> (_VBQ.bit_length() - 1)) & 3)


def _sc_logits(B, P, N, D, W, GCH, UNROLL):
    BPT = B // NW
    NCHUNK = BPT // W
    NBUF = 2
    WP, WN = W * P, W * N
    DW = D // 2

    mesh = plsc.VectorSubcoreMesh(core_axis_name="c", subcore_axis_name="s")

    scratch = []
    for _ in range(NBUF):
        scratch += [
            pltpu.VMEM((W,), jnp.int32),
            pltpu.VMEM((WP,), jnp.int32),
            pltpu.VMEM((WN,), jnp.int32),
            pltpu.VMEM((W, DW), jnp.uint32),
            pltpu.VMEM((WP, DW), jnp.uint32),
            pltpu.VMEM((WN, DW), jnp.uint32),
            pltpu.SemaphoreType.DMA,
            pltpu.SemaphoreType.DMA,
        ]
    PN = P + N
    scratch += [
        pltpu.VMEM((BPT * PN,), jnp.float32),
    ]

    @functools.partial(
        pl.kernel,
        out_type=jax.ShapeDtypeStruct((B * PN,), jnp.float32),
        mesh=mesh,
        scratch_types=scratch,
        compiler_params=pltpu.CompilerParams(needs_layout_passes=False,
                                             use_tc_tiling_on_sc=False),
    )
    def k(inp_ids, near_ids, neg_ids, table, out_all, *s):
        bufs = [s[i * 8:(i + 1) * 8] for i in range(NBUF)]
        lb_all = s[NBUF * 8]
        wid = lax.axis_index("s") * NC + lax.axis_index("c")
        base = wid * BPT
        LAST = NCHUNK - 1

        def fire_idx(c, r):
            idx_i, idx_p, idx_n = bufs[r][0:3]
            isem = bufs[r][7]
            b0 = base + c * W
            pltpu.async_copy(inp_ids.at[pl.ds(b0, W)], idx_i, isem)
            pltpu.async_copy(near_ids.at[pl.ds(b0 * P, WP)], idx_p, isem)
            pltpu.async_copy(neg_ids.at[pl.ds(b0 * N, WN)], idx_n, isem)

        def wait_idx(r):
            idx_i, idx_p, idx_n = bufs[r][0:3]
            isem = bufs[r][7]
            pltpu.make_async_copy(inp_ids.at[pl.ds(0, W)], idx_i, isem).wait()
            pltpu.make_async_copy(near_ids.at[pl.ds(0, WP)], idx_p, isem).wait()
            pltpu.make_async_copy(neg_ids.at[pl.ds(0, WN)], idx_n, isem).wait()

        def fire_gathers(r):
            idx_i, idx_p, idx_n, rows_i, rows_p, rows_n, gsem, _ = bufs[r]
            pltpu.async_copy(table.at[idx_i], rows_i, gsem)
            for o in range(0, WP, GCH):
                pltpu.async_copy(table.at[idx_p.at[pl.ds(o, GCH)]],
                                 rows_p.at[pl.ds(o, GCH)], gsem)
            for o in range(0, WN, GCH):
                pltpu.async_copy(table.at[idx_n.at[pl.ds(o, GCH)]],
                                 rows_n.at[pl.ds(o, GCH)], gsem)

        def drain_gathers(r):
            idx_i, idx_p, idx_n, rows_i, rows_p, rows_n, gsem, _ = bufs[r]
            pltpu.make_async_copy(table.at[idx_i], rows_i, gsem).wait()
            for o in range(0, WP, GCH):
                pltpu.make_async_copy(table.at[idx_p.at[pl.ds(o, GCH)]],
                                      rows_p.at[pl.ds(o, GCH)], gsem).wait()
            for o in range(0, WN, GCH):
                pltpu.make_async_copy(table.at[idx_n.at[pl.ds(o, GCH)]],
                                      rows_n.at[pl.ds(o, GCH)], gsem).wait()

        lane = lax.iota(jnp.int32, LANES)
        last_lane = lane == (LANES - 1)

        def unpack(w):
            return (plsc.bitcast(w << 16, jnp.float32),
                    plsc.bitcast(w & jnp.uint32(_HI), jnp.float32))

        def compute(c, r):
            rows_i, rows_p, rows_n = bufs[r][3:6]
            for b in range(W):
                ivecs = []
                for j in range(DW // LANES):
                    ivecs += unpack(rows_i[b, pl.ds(j * LANES, LANES)])

                def dots(K, rows, out, obase):
                    @plsc.parallel_loop(0, K, 1, unroll=UNROLL)
                    def _(kk):
                        row = b * K + kk
                        acc = None
                        for j in range(DW // LANES):
                            a, hi = unpack(rows[row, pl.ds(j * LANES, LANES)])
                            t = a * ivecs[2 * j] + hi * ivecs[2 * j + 1]
                            acc = t if acc is None else acc + t
                        s_ = plsc.cumsum(acc)
                        oidx = jnp.full((LANES,), obase + kk, jnp.int32)
                        plsc.store_scatter(out, [oidx], s_, mask=last_lane)

                dots(P, rows_p, lb_all, (c * W + b) * PN)
                dots(N, rows_n, lb_all, (c * W + b) * PN + P)

        fire_idx(0, 0)
        wait_idx(0)
        fire_gathers(0)
        fire_idx(1, 1)

        def step(i, _):
            for r in range(NBUF):
                c = i * NBUF + r
                drain_gathers(r)
                fire_idx(jnp.minimum(c + 2, LAST), r)
                wait_idx(r ^ 1)
                fire_gathers(r ^ 1)
                compute(c, r)
            return 0

        lax.fori_loop(0, NCHUNK // NBUF, step, 0)
        drain_gathers(0)
        wait_idx(1)

        pltpu.sync_copy(lb_all, out_all.at[pl.ds(base * PN, BPT * PN)])

    return k


def _tc_loss(lb, P):
    B, PN = lb.shape
    BLK = 2048

    def body(lb_ref, out_ref):
        x = lb_ref[...]

        def lsig(y):
            return jnp.minimum(y, 0.0) - jnp.log1p(jnp.exp(-jnp.abs(y)))
        out_ref[...] = -(lsig(x[:, :P]).sum(axis=1)
                         + lsig(-x[:, P:]).sum(axis=1))

    return pl.pallas_call(
        body,
        grid=(B // BLK,),
        in_specs=[pl.BlockSpec((BLK, PN), lambda i: (i, 0))],
        out_specs=pl.BlockSpec((BLK,), lambda i: (i,)),
        out_shape=jax.ShapeDtypeStruct((B,), jnp.float32),
    )(lb)


def kernel(input_wordids, near_wordids, neg_wordids, input_weight):
    B, P = near_wordids.shape
    N = neg_wordids.shape[1]
    V, D = input_weight.shape
    W = 8
    GCH = 80
    UNROLL = 5

    ids = _map_ids(input_wordids.astype(jnp.int32))
    near = _map_ids(near_wordids.reshape(B * P).astype(jnp.int32))
    neg = _map_ids(neg_wordids.reshape(B * N).astype(jnp.int32))
    table_pk = _tc_pack_table(input_weight)

    lb = _sc_logits(B, P, N, D, W, GCH, UNROLL)(ids, near, neg, table_pk)
    return _tc_loss(lb.reshape(B, P + N), P)


# speedup vs baseline: 1.2159x; 1.0403x over previous
import functools

import jax
import jax.numpy as jnp
from jax import lax
from jax.experimental import pallas as pl
from jax.experimental.pallas import tpu as pltpu
from jax.experimental.pallas import tpu_sc as plsc

NC = 2
NS = 16
NW = NC * NS
LANES = 16

_VB = 32768
_VBQ = _VB // 4
_HI = 0xFFFF0000


def _tc_pack_table(table):
    V, D = table.shape
    grid = -(-V // _VB)

    def body(in_ref, out_ref):
        x = in_ref[...]
        u = lax.bitcast_convert_type(x, jnp.uint32)
        rb = (u + jnp.uint32(0x7FFF) + ((u >> 16) & jnp.uint32(1))) >> 16
        w = rb[:D // 2, :] | (rb[D // 2:, :] << 16)
        wt = w.T
        out_ref[...] = jnp.concatenate(
            [wt[k * _VBQ:(k + 1) * _VBQ] for k in range(4)], axis=1)

    out = pl.pallas_call(
        body,
        grid=(grid,),
        in_specs=[pl.BlockSpec((D, _VB), lambda i: (0, i))],
        out_specs=pl.BlockSpec((_VBQ, 2 * D), lambda i: (i, 0)),
        out_shape=jax.ShapeDtypeStruct((grid * _VBQ, 2 * D), jnp.uint32),
    )(table.T)
    return out.reshape(grid * _VB, D // 2)


def _map_ids(v):
    return (v & -_VB) + ((v & (_VBQ - 1)) << 2) + ((v >> (_VBQ.bit_length() - 1)) & 3)


def _sc_logits(B, P, N, D, W, GCH, UNROLL):
    BPT = B // NW
    NCHUNK = BPT // W
    NBUF = 2
    PN = P + N
    WK = W * PN
    DW = D // 2

    mesh = plsc.VectorSubcoreMesh(core_axis_name="c", subcore_axis_name="s")

    scratch = []
    for _ in range(NBUF):
        scratch += [
            pltpu.VMEM((W,), jnp.int32),
            pltpu.VMEM((WK,), jnp.int32),
            pltpu.VMEM((W, DW), jnp.uint32),
            pltpu.VMEM((WK, DW), jnp.uint32),
            pltpu.SemaphoreType.DMA,
            pltpu.SemaphoreType.DMA,
        ]
    scratch += [
        pltpu.VMEM((BPT * PN,), jnp.float32),
    ]

    @functools.partial(
        pl.kernel,
        out_type=jax.ShapeDtypeStruct((B * PN,), jnp.float32),
        mesh=mesh,
        scratch_types=scratch,
        compiler_params=pltpu.CompilerParams(needs_layout_passes=False,
                                             use_tc_tiling_on_sc=False),
    )
    def k(inp_ids, ctx_ids, table, out_all, *s):
        bufs = [s[i * 6:(i + 1) * 6] for i in range(NBUF)]
        lb_all = s[NBUF * 6]
        wid = lax.axis_index("s") * NC + lax.axis_index("c")
        base = wid * BPT
        LAST = NCHUNK - 1

        def fire_idx(c, r):
            idx_i, idx_c = bufs[r][0:2]
            isem = bufs[r][5]
            b0 = base + c * W
            pltpu.async_copy(inp_ids.at[pl.ds(b0, W)], idx_i, isem)
            pltpu.async_copy(ctx_ids.at[pl.ds(b0 * PN, WK)], idx_c, isem)

        def wait_idx(r):
            idx_i, idx_c = bufs[r][0:2]
            isem = bufs[r][5]
            pltpu.make_async_copy(inp_ids.at[pl.ds(0, W)], idx_i, isem).wait()
            pltpu.make_async_copy(ctx_ids.at[pl.ds(0, WK)], idx_c, isem).wait()

        def fire_gathers(r):
            idx_i, idx_c, rows_i, rows_c, gsem, _ = bufs[r]
            pltpu.async_copy(table.at[idx_i], rows_i, gsem)
            for o in range(0, WK, GCH):
                pltpu.async_copy(table.at[idx_c.at[pl.ds(o, GCH)]],
                                 rows_c.at[pl.ds(o, GCH)], gsem)

        def drain_gathers(r):
            idx_i, idx_c, rows_i, rows_c, gsem, _ = bufs[r]
            pltpu.make_async_copy(table.at[idx_i], rows_i, gsem).wait()
            for o in range(0, WK, GCH):
                pltpu.make_async_copy(table.at[idx_c.at[pl.ds(o, GCH)]],
                                      rows_c.at[pl.ds(o, GCH)], gsem).wait()

        lane = lax.iota(jnp.int32, LANES)
        last_lane = lane == (LANES - 1)

        def unpack(w):
            return (plsc.bitcast(w << 16, jnp.float32),
                    plsc.bitcast(w & jnp.uint32(_HI), jnp.float32))

        def compute(c, r):
            rows_i, rows_c = bufs[r][2:4]
            for b in range(W):
                ivecs = []
                for j in range(DW // LANES):
                    ivecs += unpack(rows_i[b, pl.ds(j * LANES, LANES)])

                def dots(K, rows, rbase, out, obase):
                    @plsc.parallel_loop(0, K, 1, unroll=UNROLL)
                    def _(kk):
                        row = rbase + kk
                        acc = None
                        for j in range(DW // LANES):
                            a, hi = unpack(rows[row, pl.ds(j * LANES, LANES)])
                            t = a * ivecs[2 * j] + hi * ivecs[2 * j + 1]
                            acc = t if acc is None else acc + t
                        s_ = plsc.cumsum(acc)
                        oidx = jnp.full((LANES,), obase + kk, jnp.int32)
                        plsc.store_scatter(out, [oidx], s_, mask=last_lane)

                dots(P, rows_c, b * PN, lb_all, (c * W + b) * PN)
                dots(N, rows_c, b * PN + P, lb_all, (c * W + b) * PN + P)

        fire_idx(0, 0)
        wait_idx(0)
        fire_gathers(0)
        fire_idx(1, 1)

        def step(i, _):
            for r in range(NBUF):
                c = i * NBUF + r
                drain_gathers(r)
                fire_idx(jnp.minimum(c + 2, LAST), r)
                wait_idx(r ^ 1)
                fire_gathers(r ^ 1)
                compute(c, r)
            return 0

        lax.fori_loop(0, NCHUNK // NBUF, step, 0)
        drain_gathers(0)
        wait_idx(1)

        pltpu.sync_copy(lb_all, out_all.at[pl.ds(base * PN, BPT * PN)])

    return k


def _tc_loss(lb, P):
    B, PN = lb.shape
    BLK = 2048

    def body(lb_ref, out_ref):
        x = lb_ref[...]

        def lsig(y):
            return jnp.minimum(y, 0.0) - jnp.log1p(jnp.exp(-jnp.abs(y)))
        out_ref[...] = -(lsig(x[:, :P]).sum(axis=1)
                         + lsig(-x[:, P:]).sum(axis=1))

    return pl.pallas_call(
        body,
        grid=(B // BLK,),
        in_specs=[pl.BlockSpec((BLK, PN), lambda i: (i, 0))],
        out_specs=pl.BlockSpec((BLK,), lambda i: (i,)),
        out_shape=jax.ShapeDtypeStruct((B,), jnp.float32),
    )(lb)


def kernel(input_wordids, near_wordids, neg_wordids, input_weight):
    B, P = near_wordids.shape
    N = neg_wordids.shape[1]
    V, D = input_weight.shape
    W = 8
    GCH = 80
    UNROLL = 5

    ids = _map_ids(input_wordids.astype(jnp.int32))
    ctx = _map_ids(jnp.concatenate([near_wordids, neg_wordids], axis=1)
                   .reshape(B * (P + N)).astype(jnp.int32))
    table_pk = _tc_pack_table(input_weight)

    lb = _sc_logits(B, P, N, D, W, GCH, UNROLL)(ids, ctx, table_pk)
    return _tc_loss(lb.reshape(B, P + N), P)
